# baseline (speedup 1.0000x reference)
import functools

import jax
import jax.numpy as jnp
from jax.experimental import pallas as pl


def _pe_add_block(x_ref, pe_ref, o_ref):
    o_ref[...] = x_ref[...] + pe_ref[...]


def _pe_add_part(x_flat, pe, nbatch, batch0, block_s=2048):
    R, D = x_flat.shape
    S = pe.shape[0]
    nsb = S // block_s
    grid = (nsb, nbatch)
    return pl.pallas_call(
        _pe_add_block,
        grid=grid,
        in_specs=[
            pl.BlockSpec((block_s, D), lambda s, b: ((b + batch0) * nsb + s, 0)),
            pl.BlockSpec((block_s, D), lambda s, b: (s, 0)),
        ],
        out_specs=pl.BlockSpec((block_s, D), lambda s, b: (b * nsb + s, 0)),
        out_shape=jax.ShapeDtypeStruct((nbatch * S, D), x_flat.dtype),
    )(x_flat, pe)


def kernel(x, pe_table):
    B, S, D = x.shape
    x_flat = x.reshape(B * S, D)
    pe = pe_table[:S]
    out_a = _pe_add_part(x_flat, pe, B - 1, 0)
    out_b = _pe_add_part(x_flat, pe, 1, B - 1)
    out = jnp.concatenate([out_a, out_b], axis=0)
    return out.reshape(B, S, D)


# speedup vs baseline: 2.0473x; 2.0473x over previous
import functools

import jax
import jax.numpy as jnp
from jax.experimental import pallas as pl


def _pe_add_block(x_ref, pe_ref, o_ref):
    o_ref[...] = x_ref[...] + pe_ref[...]


@functools.partial(jax.jit, static_argnames=("block_s", "batch"))
def _pe_add(x_flat, pe, block_s=2048, batch=4):
    R, D = x_flat.shape
    S = pe.shape[0]
    nsb = S // block_s
    grid = (nsb, batch)
    return pl.pallas_call(
        _pe_add_block,
        grid=grid,
        in_specs=[
            pl.BlockSpec((block_s, D), lambda s, b: (b * nsb + s, 0)),
            pl.BlockSpec((block_s, D), lambda s, b: (s, 0)),
        ],
        out_specs=pl.BlockSpec((block_s, D), lambda s, b: (b * nsb + s, 0)),
        out_shape=jax.ShapeDtypeStruct((R, D), x_flat.dtype),
    )(x_flat, pe)


def kernel(x, pe_table):
    B, S, D = x.shape
    out = _pe_add(x.reshape(B * S, D), pe_table[:S], batch=B)
    return out.reshape(B, S, D)
